# baseline (speedup 1.0000x reference)
import functools

import jax
import jax.numpy as jnp
from jax import lax
from jax.experimental import pallas as pl
from jax.experimental.pallas import tpu as pltpu
from jax.experimental.pallas import tpu_sc as plsc

F = 26
V = 100000
E = 16
A = 8
B = 16384
NPAIR = F * (F - 1) // 2

NC, NS = 2, 16
NW = NC * NS
NROWS = B * F
NCOMP = F * E
COMP_PER_W = NCOMP // NW
BCH = 8192
NBCH = B // BCH


@functools.cache
def _make_sc_gather(nb):
    mesh = plsc.VectorSubcoreMesh(core_axis_name="c", subcore_axis_name="s")
    bch = min(nb, BCH)
    nbch = nb // bch

    @functools.partial(
        pl.kernel,
        out_type=jax.ShapeDtypeStruct((NCOMP, nb), jnp.float32),
        mesh=mesh,
        scratch_types=[
            pltpu.VMEM((V,), jnp.float32),
            pltpu.VMEM((bch,), jnp.int32),
            pltpu.VMEM((bch,), jnp.float32),
            pltpu.SemaphoreType.DMA,
        ],
        compiler_params=pltpu.CompilerParams(needs_layout_passes=False),
    )
    def _sc_gather(tbl_hbm, xt_hbm, out_hbm, tbl_v, idx_v, out_v, sem):
        wid = lax.axis_index("s") * NC + lax.axis_index("c")
        r0 = wid * COMP_PER_W

        def row_loop(k, c):
            r = r0 + k
            f = r // E
            pltpu.sync_copy(tbl_hbm.at[f, r % E], tbl_v)

            def chunk(j, c2):
                pltpu.sync_copy(xt_hbm.at[f, pl.ds(j * bch, bch)], idx_v)

                def grp(g, c3):
                    iv = idx_v[pl.ds(g * 16, 16)]
                    out_v[pl.ds(g * 16, 16)] = plsc.load_gather(tbl_v, [iv])
                    return c3

                lax.fori_loop(0, bch // 16, grp, 0)
                pltpu.sync_copy(out_v, out_hbm.at[r, pl.ds(j * bch, bch)])
                return c2

            lax.fori_loop(0, nbch, chunk, 0)
            return c

        lax.fori_loop(0, COMP_PER_W, row_loop, 0)

    return _sc_gather


BB = 256
NP13 = 13
PW = 512
IW = NP13 * PW
SW = 416


def _afm_body(x_ref, emb_ref, wt_ref, wt2_ref, abt_ref, ht_ref, e32_ref,
              es_ref, mask_ref, pp_ref, ww_ref, wb_ref, out_ref):
    e2 = emb_ref[...]
    pieces = []
    for p in range(NP13):
        d = (p + 1) * E
        rot = jnp.concatenate([e2[:, d:], e2[:, :d]], axis=1)
        pieces.append(e2 * rot)

    score_gs = []
    for p in range(NP13):
        for half in range(2):
            sl = (pieces[p][:, :256] if half == 0 else pieces[p][:, 256:])
            wmat = wt_ref[...] if half == 0 else wt2_ref[...]
            att_g = jnp.maximum(
                jnp.dot(sl, wmat, preferred_element_type=jnp.float32)
                + abt_ref[...], 0.0)
            score_gs.append(
                jnp.dot(att_g, ht_ref[...],
                        preferred_element_type=jnp.float32))
    score = jnp.concatenate(score_gs, axis=1) + mask_ref[...]

    m = jnp.max(score, axis=1, keepdims=True)
    ex = jnp.exp(score - m)
    w = ex / jnp.sum(ex, axis=1, keepdims=True)

    att_out = jnp.zeros((BB, E), jnp.float32)
    for p in range(NP13):
        wexp_p = jnp.dot(w[:, 256 * p:256 * (p + 1)], e32_ref[...],
                         preferred_element_type=jnp.float32)
        u_p = pieces[p] * wexp_p
        att_out = att_out + jnp.dot(u_p, es_ref[...],
                                    preferred_element_type=jnp.float32)
    afm = jnp.sum(att_out * pp_ref[...], axis=1)
    wide = jnp.maximum(
        jnp.sum(x_ref[...] * ww_ref[...], axis=1) + wb_ref[0, 0], 0.0)
    out_ref[...] = jax.nn.sigmoid(wide + afm).reshape(BB, 1)


def _afm_tc(X, emb2, wt, wt2, abt, ht, e32, es, mask, pp_row, ww, wb2):
    nb = X.shape[0]
    nblk = nb // BB
    full = lambda shp: pl.BlockSpec(shp, lambda i: tuple(0 for _ in shp))
    return pl.pallas_call(
        _afm_body,
        grid=(nblk,),
        in_specs=[
            pl.BlockSpec((BB, F), lambda i: (i, 0)),
            pl.BlockSpec((BB, F * E), lambda i: (i, 0)),
            full((256, 128)),
            full((160, 128)),
            full((1, 128)),
            full((128, 128)),
            full((256, F * E)),
            full((F * E, E)),
            full((1, 26 * 128)),
            full((1, E)),
            full((1, F)),
            full((1, 1)),
        ],
        out_specs=pl.BlockSpec((BB, 1), lambda i: (i, 0)),
        out_shape=jax.ShapeDtypeStruct((nb, 1), jnp.float32),
    )(X, emb2, wt, wt2, abt, ht, e32, es, mask, pp_row, ww, wb2)


def kernel(X, tables, attention_W, attention_b, projection_h, projection_p,
           wide_W, wide_b):
    XT = X.astype(jnp.int32).T
    tblT = tables.transpose(0, 2, 1)

    eye16 = jnp.eye(E, dtype=jnp.float32)
    wt = jnp.kron(eye16, attention_W)
    wt2 = wt[:160]
    abt = jnp.tile(attention_b.reshape(1, A), (1, E))
    ht = jnp.pad(jnp.kron(eye16, projection_h.reshape(A, 1)),
                 ((0, 0), (0, 112)))
    lane416 = jnp.arange(F * E)
    c = jnp.arange(256)[:, None]
    fld = lane416[None, :] // E
    e32 = (((c < 16) & (fld == c))
           | ((c >= 128) & (c < 138) & (fld == c - 112))
           ).astype(jnp.float32)
    es = (lane416[:, None] % E
          == jnp.arange(E)[None, :]).astype(jnp.float32)
    l = jnp.arange(26 * 128)
    g = l // 128
    cc = l % 128
    p_ = g // 2
    f_ = 16 * (g % 2) + cc
    valid = (cc < 16) & (f_ < F) & ((p_ < NP13 - 1) | (f_ < NP13))
    mask = jnp.where(valid, 0.0, -1e30).astype(jnp.float32).reshape(1, -1)

    nsplit = 2
    hb = B // nsplit
    outs = []
    for h in range(nsplit):
        xt_h = XT[:, h * hb:(h + 1) * hb]
        embT_h = _make_sc_gather(hb)(tblT, xt_h)
        out_h = _afm_tc(X[h * hb:(h + 1) * hb], embT_h.T, wt, wt2, abt, ht,
                        e32, es, mask, projection_p.reshape(1, E), wide_W,
                        wide_b.reshape(1, 1))
        outs.append(out_h)
    return jnp.concatenate(outs, axis=0).reshape(B)


# speedup vs baseline: 1.3428x; 1.3428x over previous
import functools

import jax
import jax.numpy as jnp
from jax import lax
from jax.experimental import pallas as pl
from jax.experimental.pallas import tpu as pltpu
from jax.experimental.pallas import tpu_sc as plsc

F = 26
V = 100000
E = 16
A = 8
B = 16384
NPAIR = F * (F - 1) // 2

NC, NS = 2, 16
NW = NC * NS
NROWS = B * F
NCOMP = F * E
COMP_PER_W = NCOMP // NW
BCH = 8192
NBCH = B // BCH


@functools.cache
def _make_sc_gather(nb):
    mesh = plsc.VectorSubcoreMesh(core_axis_name="c", subcore_axis_name="s")
    bch = min(nb, BCH)
    nbch = nb // bch

    @functools.partial(
        pl.kernel,
        out_type=jax.ShapeDtypeStruct((NCOMP, nb), jnp.float32),
        mesh=mesh,
        scratch_types=[
            pltpu.VMEM((V,), jnp.float32),
            pltpu.VMEM((bch,), jnp.int32),
            pltpu.VMEM((bch,), jnp.float32),
            pltpu.SemaphoreType.DMA,
        ],
        compiler_params=pltpu.CompilerParams(needs_layout_passes=False),
    )
    def _sc_gather(tbl_hbm, xt_hbm, out_hbm, tbl_v, idx_v, out_v, sem):
        wid = lax.axis_index("s") * NC + lax.axis_index("c")
        r0 = wid * COMP_PER_W

        def row_loop(k, c):
            r = r0 + k
            f = r // E
            pltpu.sync_copy(tbl_hbm.at[f, r % E], tbl_v)

            def chunk(j, c2):
                pltpu.sync_copy(xt_hbm.at[f, pl.ds(j * bch, bch)], idx_v)

                def grp(g, c3):
                    iv = idx_v[pl.ds(g * 16, 16)]
                    out_v[pl.ds(g * 16, 16)] = plsc.load_gather(tbl_v, [iv])
                    return c3

                lax.fori_loop(0, bch // 16, grp, 0)
                pltpu.sync_copy(out_v, out_hbm.at[r, pl.ds(j * bch, bch)])
                return c2

            lax.fori_loop(0, nbch, chunk, 0)
            return c

        lax.fori_loop(0, COMP_PER_W, row_loop, 0)

    return _sc_gather


BB = 256
NP13 = 13
PW = 512
IW = NP13 * PW
SW = 416


def _afm_body(x_ref, emb_ref, wt_ref, wt2_ref, abt_ref, ht_ref, e32_ref,
              es_ref, mask_ref, pp_ref, ww_ref, wb_ref, out_ref):
    e2 = emb_ref[...]
    pieces = []
    for p in range(NP13):
        d = (p + 1) * E
        rot = jnp.concatenate([e2[:, d:], e2[:, :d]], axis=1)
        pieces.append(e2 * rot)

    att0 = [jnp.maximum(
        jnp.dot(pieces[p][:, :256], wt_ref[...],
                preferred_element_type=jnp.float32) + abt_ref[...], 0.0)
        for p in range(NP13)]
    att1 = [jnp.maximum(
        jnp.dot(pieces[p][:, 256:], wt2_ref[...],
                preferred_element_type=jnp.float32) + abt_ref[...], 0.0)
        for p in range(NP13)]
    sc0 = [jnp.dot(a, ht_ref[...], preferred_element_type=jnp.float32)
           for a in att0]
    sc1 = [jnp.dot(a, ht_ref[...], preferred_element_type=jnp.float32)
           for a in att1]
    score_gs = []
    for p in range(NP13):
        score_gs.append(sc0[p])
        score_gs.append(sc1[p])
    score = jnp.concatenate(score_gs, axis=1) + mask_ref[...]

    m = jnp.max(score, axis=1, keepdims=True)
    ex = jnp.exp(score - m)
    w = ex / jnp.sum(ex, axis=1, keepdims=True)

    att_out = jnp.zeros((BB, E), jnp.float32)
    for p in range(NP13):
        wexp_p = jnp.dot(w[:, 256 * p:256 * (p + 1)], e32_ref[...],
                         preferred_element_type=jnp.float32)
        u_p = pieces[p] * wexp_p
        att_out = att_out + jnp.dot(u_p, es_ref[...],
                                    preferred_element_type=jnp.float32)
    afm = jnp.sum(att_out * pp_ref[...], axis=1)
    wide = jnp.maximum(
        jnp.sum(x_ref[...] * ww_ref[...], axis=1) + wb_ref[0, 0], 0.0)
    out_ref[...] = jax.nn.sigmoid(wide + afm).reshape(BB, 1)


def _afm_tc(X, emb2, wt, wt2, abt, ht, e32, es, mask, pp_row, ww, wb2):
    nb = X.shape[0]
    nblk = nb // BB
    full = lambda shp: pl.BlockSpec(shp, lambda i: tuple(0 for _ in shp))
    return pl.pallas_call(
        _afm_body,
        grid=(nblk,),
        in_specs=[
            pl.BlockSpec((BB, F), lambda i: (i, 0)),
            pl.BlockSpec((BB, F * E), lambda i: (i, 0)),
            full((256, 128)),
            full((160, 128)),
            full((1, 128)),
            full((128, 128)),
            full((256, F * E)),
            full((F * E, E)),
            full((1, 26 * 128)),
            full((1, E)),
            full((1, F)),
            full((1, 1)),
        ],
        out_specs=pl.BlockSpec((BB, 1), lambda i: (i, 0)),
        out_shape=jax.ShapeDtypeStruct((nb, 1), jnp.float32),
    )(X, emb2, wt, wt2, abt, ht, e32, es, mask, pp_row, ww, wb2)


def kernel(X, tables, attention_W, attention_b, projection_h, projection_p,
           wide_W, wide_b):
    XT = X.astype(jnp.int32).T
    tblT = tables.transpose(0, 2, 1)

    eye16 = jnp.eye(E, dtype=jnp.float32)
    wt = jnp.kron(eye16, attention_W)
    wt2 = wt[:160]
    abt = jnp.tile(attention_b.reshape(1, A), (1, E))
    ht = jnp.pad(jnp.kron(eye16, projection_h.reshape(A, 1)),
                 ((0, 0), (0, 112)))
    lane416 = jnp.arange(F * E)
    c = jnp.arange(256)[:, None]
    fld = lane416[None, :] // E
    e32 = (((c < 16) & (fld == c))
           | ((c >= 128) & (c < 138) & (fld == c - 112))
           ).astype(jnp.float32)
    es = (lane416[:, None] % E
          == jnp.arange(E)[None, :]).astype(jnp.float32)
    l = jnp.arange(26 * 128)
    g = l // 128
    cc = l % 128
    p_ = g // 2
    f_ = 16 * (g % 2) + cc
    valid = (cc < 16) & (f_ < F) & ((p_ < NP13 - 1) | (f_ < NP13))
    mask = jnp.where(valid, 0.0, -1e30).astype(jnp.float32).reshape(1, -1)

    nsplit = 2
    hb = B // nsplit
    outs = []
    for h in range(nsplit):
        xt_h = XT[:, h * hb:(h + 1) * hb]
        embT_h = _make_sc_gather(hb)(tblT, xt_h)
        out_h = _afm_tc(X[h * hb:(h + 1) * hb], embT_h.T, wt, wt2, abt, ht,
                        e32, es, mask, projection_p.reshape(1, E), wide_W,
                        wide_b.reshape(1, 1))
        outs.append(out_h)
    return jnp.concatenate(outs, axis=0).reshape(B)
